# baseline (speedup 1.0000x reference)
import functools
import jax
import jax.numpy as jnp
from jax import lax
from jax.experimental import pallas as pl
from jax.experimental.pallas import tpu as pltpu
from jax.experimental.pallas import tpu_sc as plsc

N_NODES = 10000
N_EDGES = 320000
IN_DIM = 128
OUT_DIM = 128
NUM_HEADS = 8
HEAD_DIM = OUT_DIM // NUM_HEADS

NC = 2
NS = 16
L = 16

NW = NC * NS
EPT = N_EDGES // NW
B = 16
NBLK = EPT // B


def _node_proj_body(x_ref, qw_ref, kvw_ref, qb_ref, kvb_ref,
                    q_ref, kv_ref):
    xb = x_ref[...]
    q_ref[...] = jnp.dot(xb, qw_ref[...],
                         preferred_element_type=jnp.float32) + qb_ref[...]
    kv_ref[...] = (jnp.dot(xb, kvw_ref[...],
                           preferred_element_type=jnp.float32)
                   + kvb_ref[...]).astype(jnp.bfloat16)


def _node_proj(x, Qw, Qb, Kw, Kb, Vw, Vb):
    blk = 2000
    grid = (N_NODES // blk,)
    perm = _lane_perm()
    KVw = jnp.concatenate([Kw[:, perm], Vw[:, perm]], axis=1)
    KVb = jnp.concatenate([Kb[perm], Vb[perm]])
    return pl.pallas_call(
        _node_proj_body,
        grid=grid,
        in_specs=[pl.BlockSpec((blk, IN_DIM), lambda i: (i, 0)),
                  pl.BlockSpec((IN_DIM, OUT_DIM), lambda i: (0, 0)),
                  pl.BlockSpec((IN_DIM, 2 * OUT_DIM), lambda i: (0, 0)),
                  pl.BlockSpec((1, OUT_DIM), lambda i: (0, 0)),
                  pl.BlockSpec((1, 2 * OUT_DIM), lambda i: (0, 0))],
        out_specs=[pl.BlockSpec((blk, OUT_DIM), lambda i: (i, 0)),
                   pl.BlockSpec((blk, 2 * OUT_DIM), lambda i: (i, 0))],
        out_shape=[jax.ShapeDtypeStruct((N_NODES, OUT_DIM), jnp.float32),
                   jax.ShapeDtypeStruct((N_NODES, 2 * OUT_DIM), jnp.bfloat16)],
    )(x, Qw, KVw, Qb.reshape(1, -1), KVb.reshape(1, -1))


def _edge_proj_body(at_ref, ew_ref, eb_ref, e_ref):
    ef = lax.dot_general(
        at_ref[...], ew_ref[...],
        (((0,), (0,)), ((), ())),
        preferred_element_type=jnp.float32) + eb_ref[...]
    e_ref[...] = ef.astype(jnp.bfloat16)


def _edge_proj(attr_t, Ew, Eb):
    blk = 6400
    grid = (N_EDGES // blk,)
    dim_edge = attr_t.shape[0]
    return pl.pallas_call(
        _edge_proj_body,
        grid=grid,
        in_specs=[pl.BlockSpec((dim_edge, blk), lambda i: (0, i)),
                  pl.BlockSpec((dim_edge, OUT_DIM), lambda i: (0, 0)),
                  pl.BlockSpec((1, OUT_DIM), lambda i: (0, 0))],
        out_specs=pl.BlockSpec((blk, OUT_DIM), lambda i: (i, 0)),
        out_shape=jax.ShapeDtypeStruct((N_EDGES, OUT_DIM), jnp.bfloat16),
    )(attr_t, Ew, Eb.reshape(1, -1))


def _edge_kernel(kv_hbm, q_hbm, e_hbm, src_hbm, dst_hbm,
                 out_hbm,
                 src_all, dst_all, kvr0, qr0, er0, kvr1, qr1, er1,
                 mr0, mr1, wv,
                 sem_g0, sem_g1, sem_s0, sem_s1):
    c = lax.axis_index("c")
    s = lax.axis_index("s")
    wid = s * NC + c

    r0 = s * 624
    rows_per_tile = 640

    pltpu.sync_copy(src_hbm.at[wid], src_all)
    pltpu.sync_copy(dst_hbm.at[wid], dst_all)

    z16 = jnp.zeros((HEAD_DIM,), jnp.float32)
    for i in range(B):
        for h in range(NUM_HEADS):
            dsh = pl.ds(h * HEAD_DIM, HEAD_DIM)
            mr0[i, dsh] = z16
            mr1[i, dsh] = z16
    for t in range(rows_per_tile // B):
        pltpu.sync_copy(mr0, wv.at[pl.ds(r0 + t * B, B)])
    plsc.subcore_barrier()

    def issue_gathers(b, kvr, qr, er, sem):
        e0 = (wid * NBLK + b) * B
        sv = src_all[pl.ds(b * B, B)]
        dv = dst_all[pl.ds(b * B, B)]
        pltpu.async_copy(kv_hbm.at[sv], kvr, sem)
        pltpu.async_copy(q_hbm.at[dv], qr, sem)
        pltpu.async_copy(e_hbm.at[pl.ds(e0, B)], er, sem)

    def wait_gathers(kvr, qr, er, sem):
        zv = src_all[pl.ds(0, B)]
        pltpu.make_async_copy(kv_hbm.at[zv], kvr, sem).wait()
        pltpu.make_async_copy(q_hbm.at[zv], qr, sem).wait()
        pltpu.make_async_copy(e_hbm.at[pl.ds(0, B)], er, sem).wait()

    shl = jnp.full((HEAD_DIM,), 16, jnp.int32)
    msk = jnp.full((HEAD_DIM,), -65536, jnp.int32)

    def compute(kvr, qr, er, mr):
        def up(ref, i, col):
            x = plsc.bitcast(ref[i, pl.ds(col, 32)], jnp.int32)
            lo = plsc.bitcast(x << shl, jnp.float32)
            hi = plsc.bitcast(x & msk, jnp.float32)
            return lo, hi

        for i in range(B):
            for hp in range(NUM_HEADS // 2):
                kl, kh = up(kvr, i, 32 * hp)
                vl, vh = up(kvr, i, OUT_DIM + 32 * hp)
                el, eh = up(er, i, 32 * hp)
                for sub, kk, vv, ee in ((0, kl, vl, el), (1, kh, vh, eh)):
                    h = 2 * hp + sub
                    dsh = pl.ds(h * HEAD_DIM, HEAD_DIM)
                    p = kk * qr[i, dsh] * ee
                    t = jnp.clip(jnp.sum(p), -5.0, 5.0)
                    sc = jnp.exp(jnp.full((HEAD_DIM,), t, jnp.float32))
                    mr[i, dsh] = vv * sc

    def drain_scatter(mr, sem):
        pltpu.make_async_copy(mr, wv.at[dst_all[pl.ds(0, B)]], sem).wait()

    zv = dst_all[pl.ds(0, B)]
    pltpu.async_copy(mr0, wv.at[zv], sem_s0, add=True)
    pltpu.async_copy(mr1, wv.at[zv], sem_s1, add=True)
    issue_gathers(0, kvr0, qr0, er0, sem_g0)

    def pair_body(j, carry):
        b0 = 2 * j
        b1 = b0 + 1
        b2 = b0 + 2

        @pl.when(b1 < NBLK)
        def _():
            issue_gathers(b1, kvr1, qr1, er1, sem_g1)

        wait_gathers(kvr0, qr0, er0, sem_g0)
        drain_scatter(mr0, sem_s0)
        compute(kvr0, qr0, er0, mr0)
        pltpu.async_copy(mr0, wv.at[dst_all[pl.ds(b0 * B, B)]], sem_s0, add=True)

        @pl.when(b2 < NBLK)
        def _():
            issue_gathers(b2, kvr0, qr0, er0, sem_g0)

        @pl.when(b1 < NBLK)
        def _():
            wait_gathers(kvr1, qr1, er1, sem_g1)
            drain_scatter(mr1, sem_s1)
            compute(kvr1, qr1, er1, mr1)
            pltpu.async_copy(mr1, wv.at[dst_all[pl.ds(b1 * B, B)]], sem_s1, add=True)

        return carry

    lax.fori_loop(0, (NBLK + 1) // 2, pair_body, 0)
    drain_scatter(mr0, sem_s0)
    drain_scatter(mr1, sem_s1)
    plsc.subcore_barrier()

    pltpu.sync_copy(wv.at[pl.ds(r0, rows_per_tile)],
                    out_hbm.at[c, pl.ds(r0, rows_per_tile)])


@functools.partial(jax.jit, static_argnames=())
def _edge_stage(KV_h, Q_h, E, src, dst):
    mesh = plsc.VectorSubcoreMesh(core_axis_name="c", subcore_axis_name="s")
    rowbuf = pltpu.VMEM((B, OUT_DIM), jnp.float32)
    f = pl.kernel(
        _edge_kernel,
        out_type=jax.ShapeDtypeStruct((NC, N_NODES, OUT_DIM), jnp.float32),
        mesh=mesh,
        compiler_params=pltpu.CompilerParams(needs_layout_passes=False, use_tc_tiling_on_sc=False),
        scratch_types=[
            pltpu.VMEM((EPT,), jnp.int32),
            pltpu.VMEM((EPT,), jnp.int32),
            pltpu.VMEM((B, 2 * OUT_DIM), jnp.bfloat16), rowbuf,
            pltpu.VMEM((B, OUT_DIM), jnp.bfloat16),
            pltpu.VMEM((B, 2 * OUT_DIM), jnp.bfloat16), rowbuf,
            pltpu.VMEM((B, OUT_DIM), jnp.bfloat16),
            rowbuf, rowbuf,
            pltpu.VMEM_SHARED((N_NODES, OUT_DIM), jnp.float32),
            pltpu.SemaphoreType.DMA,
            pltpu.SemaphoreType.DMA,
            pltpu.SemaphoreType.DMA,
            pltpu.SemaphoreType.DMA,
        ],
    )
    return f(KV_h, Q_h, E, src, dst)


def _lane_perm():
    perm = [0] * OUT_DIM
    for p in range(NUM_HEADS // 2):
        for t in range(HEAD_DIM):
            for sbt in range(2):
                perm[32 * p + 2 * t + sbt] = 16 * (2 * p + sbt) + t
    return jnp.array(perm, dtype=jnp.int32)


def kernel(x, expander_edge_index, expander_edge_attr, batch,
           Qw, Qb, Kw, Kb, Ew, Eb, Vw, Vb):
    Q_h, KV_h = _node_proj(x, Qw, Qb, Kw, Kb, Vw, Vb)
    inv_sqrt_d = 1.0 / (HEAD_DIM ** 0.5)
    perm = _lane_perm()
    E = _edge_proj(expander_edge_attr.T,
                   (Ew * inv_sqrt_d)[:, perm], (Eb * inv_sqrt_d)[perm])
    src = expander_edge_index[0].astype(jnp.int32).reshape(NW, EPT)
    dst = expander_edge_index[1].astype(jnp.int32).reshape(NW, EPT)
    parts = _edge_stage(KV_h, Q_h, E, src, dst)
    return parts[0] + parts[1]


# speedup vs baseline: 1.3409x; 1.3409x over previous
import functools
import jax
import jax.numpy as jnp
from jax import lax
from jax.experimental import pallas as pl
from jax.experimental.pallas import tpu as pltpu
from jax.experimental.pallas import tpu_sc as plsc

N_NODES = 10000
N_EDGES = 320000
IN_DIM = 128
OUT_DIM = 128
NUM_HEADS = 8
HEAD_DIM = OUT_DIM // NUM_HEADS

NC = 2
NS = 16
L = 16

NW = NC * NS
EPT = N_EDGES // NW
B = 16
NBLK = EPT // B


def _node_proj_body(x_ref, qw_ref, kw_ref, vw_ref, qb_ref, kb_ref, vb_ref,
                    q_ref, k_ref, v_ref):
    xb = x_ref[...]
    q_ref[...] = jnp.dot(xb, qw_ref[...],
                         preferred_element_type=jnp.float32) + qb_ref[...]
    k_ref[...] = jnp.dot(xb, kw_ref[...],
                         preferred_element_type=jnp.float32) + kb_ref[...]
    v_ref[...] = jnp.dot(xb, vw_ref[...],
                         preferred_element_type=jnp.float32) + vb_ref[...]


def _node_proj(x, Qw, Qb, Kw, Kb, Vw, Vb):
    blk = 2000
    grid = (N_NODES // blk,)
    full = pl.BlockSpec((IN_DIM, OUT_DIM), lambda i: (0, 0))
    bias = pl.BlockSpec((1, OUT_DIM), lambda i: (0, 0))
    rows = pl.BlockSpec((blk, OUT_DIM), lambda i: (i, 0))
    return pl.pallas_call(
        _node_proj_body,
        grid=grid,
        in_specs=[pl.BlockSpec((blk, IN_DIM), lambda i: (i, 0)),
                  full, full, full, bias, bias, bias],
        out_specs=[rows, rows, rows],
        out_shape=[jax.ShapeDtypeStruct((N_NODES, OUT_DIM), jnp.float32)] * 3,
    )(x, Qw, Kw, Vw, Qb.reshape(1, -1), Kb.reshape(1, -1), Vb.reshape(1, -1))


def _edge_proj_body(at_ref, ew_ref, eb_ref, e_ref):
    e_ref[...] = lax.dot_general(
        at_ref[...], ew_ref[...],
        (((0,), (0,)), ((), ())),
        preferred_element_type=jnp.float32) + eb_ref[...]


def _edge_proj(attr_t, Ew, Eb):
    blk = 6400
    grid = (N_EDGES // blk,)
    dim_edge = attr_t.shape[0]
    return pl.pallas_call(
        _edge_proj_body,
        grid=grid,
        in_specs=[pl.BlockSpec((dim_edge, blk), lambda i: (0, i)),
                  pl.BlockSpec((dim_edge, OUT_DIM), lambda i: (0, 0)),
                  pl.BlockSpec((1, OUT_DIM), lambda i: (0, 0))],
        out_specs=pl.BlockSpec((blk, OUT_DIM), lambda i: (i, 0)),
        out_shape=jax.ShapeDtypeStruct((N_EDGES, OUT_DIM), jnp.float32),
    )(attr_t, Ew, Eb.reshape(1, -1))


def _edge_kernel(k_hbm, q_hbm, v_hbm, e_hbm, src_hbm, dst_hbm,
                 out_hbm,
                 src_all, dst_all, kr0, qr0, vr0, er0, kr1, qr1, vr1, er1,
                 mr0, mr1, wv,
                 sem_g0, sem_g1, sem_s0, sem_s1):
    c = lax.axis_index("c")
    s = lax.axis_index("s")
    wid = s * NC + c

    r0 = s * 624
    rows_per_tile = 640

    pltpu.sync_copy(src_hbm.at[wid], src_all)
    pltpu.sync_copy(dst_hbm.at[wid], dst_all)

    z16 = jnp.zeros((HEAD_DIM,), jnp.float32)
    for i in range(B):
        for h in range(NUM_HEADS):
            dsh = pl.ds(h * HEAD_DIM, HEAD_DIM)
            mr0[i, dsh] = z16
            mr1[i, dsh] = z16
    for t in range(rows_per_tile // B):
        pltpu.sync_copy(mr0, wv.at[pl.ds(r0 + t * B, B)])
    plsc.subcore_barrier()

    def issue_gathers(b, kr, qr, vr, er, sem):
        e0 = (wid * NBLK + b) * B
        sv = src_all[pl.ds(b * B, B)]
        dv = dst_all[pl.ds(b * B, B)]
        pltpu.async_copy(k_hbm.at[sv], kr, sem)
        pltpu.async_copy(q_hbm.at[dv], qr, sem)
        pltpu.async_copy(v_hbm.at[sv], vr, sem)
        pltpu.async_copy(e_hbm.at[pl.ds(e0, B)], er, sem)

    def wait_gathers(kr, qr, vr, er, sem):
        zv = src_all[pl.ds(0, B)]
        pltpu.make_async_copy(k_hbm.at[zv], kr, sem).wait()
        pltpu.make_async_copy(q_hbm.at[zv], qr, sem).wait()
        pltpu.make_async_copy(v_hbm.at[zv], vr, sem).wait()
        pltpu.make_async_copy(e_hbm.at[pl.ds(0, B)], er, sem).wait()

    def compute(kr, qr, vr, er, mr):
        for i in range(B):
            for h in range(NUM_HEADS):
                dsh = pl.ds(h * HEAD_DIM, HEAD_DIM)
                p = kr[i, dsh] * qr[i, dsh] * er[i, dsh]
                t = jnp.clip(jnp.sum(p), -5.0, 5.0)
                sc = jnp.exp(jnp.full((HEAD_DIM,), t, jnp.float32))
                mr[i, dsh] = vr[i, dsh] * sc

    def drain_scatter(mr, sem):
        pltpu.make_async_copy(mr, wv.at[dst_all[pl.ds(0, B)]], sem).wait()

    zv = dst_all[pl.ds(0, B)]
    pltpu.async_copy(mr0, wv.at[zv], sem_s0, add=True)
    pltpu.async_copy(mr1, wv.at[zv], sem_s1, add=True)
    issue_gathers(0, kr0, qr0, vr0, er0, sem_g0)

    def pair_body(j, carry):
        b0 = 2 * j
        b1 = b0 + 1
        b2 = b0 + 2

        @pl.when(b1 < NBLK)
        def _():
            issue_gathers(b1, kr1, qr1, vr1, er1, sem_g1)

        wait_gathers(kr0, qr0, vr0, er0, sem_g0)
        drain_scatter(mr0, sem_s0)
        compute(kr0, qr0, vr0, er0, mr0)
        pltpu.async_copy(mr0, wv.at[dst_all[pl.ds(b0 * B, B)]], sem_s0, add=True)

        @pl.when(b2 < NBLK)
        def _():
            issue_gathers(b2, kr0, qr0, vr0, er0, sem_g0)

        @pl.when(b1 < NBLK)
        def _():
            wait_gathers(kr1, qr1, vr1, er1, sem_g1)
            drain_scatter(mr1, sem_s1)
            compute(kr1, qr1, vr1, er1, mr1)
            pltpu.async_copy(mr1, wv.at[dst_all[pl.ds(b1 * B, B)]], sem_s1, add=True)

        return carry

    lax.fori_loop(0, (NBLK + 1) // 2, pair_body, 0)
    drain_scatter(mr0, sem_s0)
    drain_scatter(mr1, sem_s1)
    plsc.subcore_barrier()

    pltpu.sync_copy(wv.at[pl.ds(r0, rows_per_tile)],
                    out_hbm.at[c, pl.ds(r0, rows_per_tile)])


@functools.partial(jax.jit, static_argnames=())
def _edge_stage(K_h, Q_h, V_h, E, src, dst):
    mesh = plsc.VectorSubcoreMesh(core_axis_name="c", subcore_axis_name="s")
    rowbuf = pltpu.VMEM((B, OUT_DIM), jnp.float32)
    f = pl.kernel(
        _edge_kernel,
        out_type=jax.ShapeDtypeStruct((NC, N_NODES, OUT_DIM), jnp.float32),
        mesh=mesh,
        compiler_params=pltpu.CompilerParams(needs_layout_passes=False, use_tc_tiling_on_sc=False),
        scratch_types=[
            pltpu.VMEM((EPT,), jnp.int32),
            pltpu.VMEM((EPT,), jnp.int32),
            rowbuf, rowbuf, rowbuf, rowbuf,
            rowbuf, rowbuf, rowbuf, rowbuf,
            rowbuf, rowbuf,
            pltpu.VMEM_SHARED((N_NODES, OUT_DIM), jnp.float32),
            pltpu.SemaphoreType.DMA,
            pltpu.SemaphoreType.DMA,
            pltpu.SemaphoreType.DMA,
            pltpu.SemaphoreType.DMA,
        ],
    )
    return f(K_h, Q_h, V_h, E, src, dst)


def kernel(x, expander_edge_index, expander_edge_attr, batch,
           Qw, Qb, Kw, Kb, Ew, Eb, Vw, Vb):
    Q_h, K_h, V_h = _node_proj(x, Qw, Qb, Kw, Kb, Vw, Vb)
    inv_sqrt_d = 1.0 / (HEAD_DIM ** 0.5)
    E = _edge_proj(expander_edge_attr.T, Ew * inv_sqrt_d, Eb * inv_sqrt_d)
    src = expander_edge_index[0].astype(jnp.int32).reshape(NW, EPT)
    dst = expander_edge_index[1].astype(jnp.int32).reshape(NW, EPT)
    parts = _edge_stage(K_h, Q_h, V_h, E, src, dst)
    return parts[0] + parts[1]
